# baseline (speedup 1.0000x reference)
import functools

import jax
import jax.numpy as jnp
from jax import lax
from jax.experimental import pallas as pl
from jax.experimental.pallas import tpu as pltpu

_INF = float(1e30)
_THRESH = float(1e29)
_BIGI = 2**30


def _body(n, m, nrow, x_ref, y_ref, bx_ref, by_ref, oi_ref, ids_ref,
          g0_ref, iso_ref, dbg_ref, gt_ref, obj_ref,
          mdist_ref, rmin_ref, rarg_ref):
    np_ = x_ref.shape[0]
    x = x_ref[:]
    y = y_ref[:]
    bx = bx_ref[:]
    by = by_ref[:]
    oi = oi_ref[:]
    ids = ids_ref[:]
    g0 = g0_ref[:]
    iso = iso_ref[:]

    rit = lax.broadcasted_iota(jnp.int32, (np_, 1), 0)
    cit = lax.broadcasted_iota(jnp.int32, (1, m), 1)
    citb = lax.broadcasted_iota(jnp.int32, (np_, m), 1)
    valid = rit < n

    match = (oi == ids) & valid
    jc = jnp.where(match, citb, -1)
    gt0 = jnp.max(jc, axis=1, keepdims=True)
    row_has = gt0 >= 0
    col_has = jnp.any(match, axis=0, keepdims=True)
    apr = (g0 >= 0) | row_has | jnp.logical_not(valid)

    dist = (x - bx) ** 2 + (y - by) ** 2
    mdist = jnp.where(apr, _INF, dist)
    mdist_ref[:] = mdist
    colmask0 = jnp.where(col_has, _INF, jnp.float32(0.0))

    md = mdist + colmask0
    rmin0 = jnp.min(md, axis=1, keepdims=True)
    rarg0 = jnp.min(jnp.where(md == rmin0, citb, _BIGI), axis=1, keepdims=True)
    rmin_ref[:] = rmin0.reshape(nrow, m)
    rarg_ref[:] = rarg0.reshape(nrow, m)

    dbg0 = jnp.where(row_has, jnp.int32(2), jnp.int32(0))
    dbg0 = dbg0 + jnp.where(iso > 0.5, jnp.int32(10), jnp.int32(0))
    dbg_ref[:] = dbg0.reshape(nrow, m)
    gt_ref[:] = gt0.reshape(nrow, m)
    obj_ref[:] = oi.reshape(nrow, m)

    riota = lax.broadcasted_iota(jnp.int32, (nrow, m), 0) * m + \
        lax.broadcasted_iota(jnp.int32, (nrow, m), 1)

    def step(_, colmask):
        def vcond(st):
            return jnp.logical_not(st[0])

        def vbody(st):
            _, _, _, _ = st
            rmin = rmin_ref[:]
            gmn = jnp.min(rmin)
            sel = rmin == gmn
            gi = jnp.min(jnp.where(sel, riota, _BIGI))
            ga = jnp.min(jnp.where(sel & (riota == gi), rarg_ref[:], _BIGI))
            live = gmn < _THRESH
            cmv = jnp.min(jnp.where(cit == ga, colmask, _INF))
            stale = live & (cmv > 0.0)

            @pl.when(stale)
            def _():
                drow = mdist_ref[pl.ds(gi, 1), :] + colmask
                nm = jnp.min(drow)
                na = jnp.min(jnp.where(drow == nm, cit, _BIGI))
                a = gi // m
                b = gi % m
                old = rmin_ref[pl.ds(a, 1), :]
                rmin_ref[pl.ds(a, 1), :] = jnp.where(cit == b, nm, old)
                olda = rarg_ref[pl.ds(a, 1), :]
                rarg_ref[pl.ds(a, 1), :] = jnp.where(cit == b, na, olda)

            return (jnp.logical_not(stale), gmn, gi, ga)

        st = lax.while_loop(vcond, vbody,
                            (jnp.bool_(False), jnp.float32(0.0),
                             jnp.int32(0), jnp.int32(0)))
        _, gmn, gi, ga = st
        do = gmn < _THRESH

        @pl.when(do)
        def _():
            a = gi // m
            b = gi % m
            isb = cit == b
            rmin_ref[pl.ds(a, 1), :] = jnp.where(
                isb, _INF, rmin_ref[pl.ds(a, 1), :])
            gt_ref[pl.ds(a, 1), :] = jnp.where(
                isb, ga, gt_ref[pl.ds(a, 1), :])
            oj = jnp.min(jnp.where(cit == ga, ids, _BIGI))
            obj_ref[pl.ds(a, 1), :] = jnp.where(
                isb, oj, obj_ref[pl.ds(a, 1), :])
            dbg_ref[pl.ds(a, 1), :] = jnp.where(
                isb, dbg_ref[pl.ds(a, 1), :] + 3, dbg_ref[pl.ds(a, 1), :])

        return jnp.where((cit == ga) & do, _INF, colmask)

    lax.fori_loop(0, m, step, colmask0)


def kernel(is_object, position, boxes, gt_idx, obj_idx, obj_ids):
    n = gt_idx.shape[0]
    m = obj_ids.shape[0]
    np_ = ((n + m - 1) // m) * m
    nrow = np_ // m
    pad = np_ - n

    x = jnp.pad(position[-1, 0, :, 0], (0, pad)).reshape(np_, 1)
    y = jnp.pad(position[-1, 0, :, 1], (0, pad)).reshape(np_, 1)
    bx = boxes[:, 0].reshape(1, m)
    by = boxes[:, 1].reshape(1, m)
    oi = jnp.pad(obj_idx.astype(jnp.int32), (0, pad),
                 constant_values=-1).reshape(np_, 1)
    ids = obj_ids.astype(jnp.int32).reshape(1, m)
    g0 = jnp.pad(gt_idx.astype(jnp.int32), (0, pad),
                 constant_values=-1).reshape(np_, 1)
    iso = jnp.pad(is_object[-1, 0, :, 0], (0, pad)).reshape(np_, 1)

    out_shape = [jax.ShapeDtypeStruct((nrow, m), jnp.int32)] * 3
    dbg, gt, obj = pl.pallas_call(
        functools.partial(_body, n, m, nrow),
        out_shape=out_shape,
        scratch_shapes=[
            pltpu.VMEM((np_, m), jnp.float32),
            pltpu.VMEM((nrow, m), jnp.float32),
            pltpu.VMEM((nrow, m), jnp.int32),
        ],
    )(x, y, bx, by, oi, ids, g0, iso)
    return (dbg.reshape(np_)[:n], gt.reshape(np_)[:n], obj.reshape(np_)[:n])
